# Initial kernel scaffold; baseline (speedup 1.0000x reference)
import jax
import jax.numpy as jnp
from jax.experimental import pallas as pl


def kernel(inputs, table):
    raise NotImplementedError("write your pallas kernel here")


# speedup vs baseline: 1.7466x; 1.7466x over previous
import functools

import jax
import jax.numpy as jnp
from jax import lax
from jax.experimental import pallas as pl
from jax.experimental.pallas import tpu as pltpu
from jax.experimental.pallas import tpu_sc as plsc

B = 16384
H = 20
D = 32
NW = 32
BPW = B // NW
TILE_B = 64
NT = BPW // TILE_B
IPT = TILE_B * H
CH = 128
NCH = IPT // CH
IDX_ROWS_PER_W = BPW * H // CH


@functools.partial(
    pl.kernel,
    mesh=plsc.VectorSubcoreMesh(core_axis_name="c", subcore_axis_name="s"),
    out_type=jax.ShapeDtypeStruct((B, D), jnp.float32),
    compiler_params=pltpu.CompilerParams(use_tc_tiling_on_sc=False),
    scratch_types=[
        pltpu.VMEM((IDX_ROWS_PER_W, CH), jnp.int32),
        pltpu.VMEM((IPT, D), jnp.float32),
        pltpu.VMEM((IPT, D), jnp.float32),
        pltpu.VMEM((BPW, D), jnp.float32),
        pltpu.SemaphoreType.DMA,
        pltpu.SemaphoreType.DMA,
    ],
)
def _emb_lookup(idx_hbm, table_hbm, out_hbm,
                idx_v, rows_a, rows_b, out_v, sem_a, sem_b):
    wid = lax.axis_index("s") * 2 + lax.axis_index("c")
    pltpu.sync_copy(idx_hbm.at[pl.ds(wid * IDX_ROWS_PER_W, IDX_ROWS_PER_W)],
                    idx_v)

    bufs = ((rows_a, sem_a), (rows_b, sem_b))

    def fire(t):
        buf, sem = bufs[t % 2]
        return [
            pltpu.async_copy(
                table_hbm.at[idx_v.at[t * NCH + k]],
                buf.at[pl.ds(k * CH, CH)],
                sem,
            )
            for k in range(NCH)
        ]

    inflight = fire(0)
    for t in range(NT):
        nxt = fire(t + 1) if t + 1 < NT else []
        for cp in inflight:
            cp.wait()
        inflight = nxt
        buf, _ = bufs[t % 2]

        def body(b, _, buf=buf, t=t):
            base = b * H
            acc0 = buf[base, pl.ds(0, 16)]
            acc1 = buf[base, pl.ds(16, 16)]
            for h in range(1, H):
                acc0 = acc0 + buf[base + h, pl.ds(0, 16)]
                acc1 = acc1 + buf[base + h, pl.ds(16, 16)]
            ob = t * TILE_B + b
            out_v[ob, pl.ds(0, 16)] = acc0 * (1.0 / H)
            out_v[ob, pl.ds(16, 16)] = acc1 * (1.0 / H)
            return 0

        lax.fori_loop(0, TILE_B, body, 0)

    pltpu.sync_copy(out_v, out_hbm.at[pl.ds(wid * BPW, BPW)])


def kernel(inputs, table):
    idx = inputs.astype(jnp.int32).reshape(B * H // CH, CH)
    return _emb_lookup(idx, table)
